# baseline (speedup 1.0000x reference)
import functools

import jax
import jax.numpy as jnp
from jax import lax
from jax.experimental import pallas as pl
from jax.experimental.pallas import tpu as pltpu
from jax.experimental.pallas import tpu_sc as plsc

_NC = 2
_NS = 16
_NW = _NC * _NS
_CH = 8
_NBUF = 2
_K = 4
_BR = 512


def _sc_gather_bf16(table, idx3):
    nw, nchunks, ch = idx3.shape
    d = table.shape[1]
    n_rows = nw * nchunks * ch
    per_w = nchunks * ch
    n_rounds = nchunks // _NBUF

    @functools.partial(
        pl.kernel,
        mesh=plsc.VectorSubcoreMesh(core_axis_name="c", subcore_axis_name="s"),
        out_type=jax.ShapeDtypeStruct((n_rows, d // 2), jnp.uint32),
        compiler_params=pltpu.CompilerParams(needs_layout_passes=False),
        scratch_types=[
            pltpu.VMEM((nchunks, ch), jnp.int32),
            pltpu.VMEM((ch, d), jnp.float32),
            pltpu.VMEM((ch, d), jnp.float32),
            pltpu.VMEM((ch, d // 2), jnp.uint32),
            pltpu.VMEM((ch, d // 2), jnp.uint32),
            pltpu.SemaphoreType.DMA,
            pltpu.SemaphoreType.DMA,
        ],
    )
    def k(table_hbm, idx_hbm, out_hbm, idx_v, rows0, rows1, o0, o1, sem0, sem1):
        wid = lax.axis_index("s") * _NC + lax.axis_index("c")
        base = wid * per_w
        pltpu.sync_copy(idx_hbm.at[wid], idx_v)
        rows = (rows0, rows1)
        outs = (o0, o1)
        sems = (sem0, sem1)
        half = d // 2

        def convert(src, dst):
            for r in range(ch):

                @plsc.parallel_loop(0, half, step=16, unroll=8)
                def _cv(w):
                    a = plsc.bitcast(src[r, pl.ds(w, 16)], jnp.uint32)
                    b = plsc.bitcast(src[r, pl.ds(half + w, 16)], jnp.uint32)
                    lo = (a + jnp.uint32(0x8000)) >> jnp.uint32(16)
                    hi = (b + jnp.uint32(0x8000)) & jnp.uint32(0xFFFF0000)
                    dst[r, pl.ds(w, 16)] = lo | hi

        for b in range(_NBUF):
            pltpu.async_copy(table_hbm.at[idx_v.at[b]], rows[b], sems[b])

        def round_body(r, carry):
            for b in range(_NBUF):
                c = r * _NBUF + b
                pltpu.make_async_copy(table_hbm.at[idx_v.at[c]], rows[b], sems[b]).wait()
                convert(rows[b], outs[b])
                pltpu.async_copy(table_hbm.at[idx_v.at[c + _NBUF]], rows[b], sems[b])
                pltpu.sync_copy(outs[b], out_hbm.at[pl.ds(base + c * ch, ch)])
            return carry

        lax.fori_loop(0, n_rounds - 1, round_body, 0)
        for b in range(_NBUF):
            c = (n_rounds - 1) * _NBUF + b
            pltpu.make_async_copy(table_hbm.at[idx_v.at[c]], rows[b], sems[b]).wait()
            convert(rows[b], outs[b])
            pltpu.sync_copy(outs[b], out_hbm.at[pl.ds(base + c * ch, ch)])

    return k(table, idx3)


def _tc_ln_add_chunk(xin, gathered, scale, gamma, beta, accum, block_off):
    n, d = xin.shape
    rows = gathered.shape[0]
    grid = (rows // _BR,)

    def body(s_ref, x_ref, g_ref, ga_ref, be_ref, *rest):
        o_ref = rest[-1]
        half = d // 2
        gu = g_ref[...]
        sc = s_ref[0]
        a = lax.bitcast_convert_type(gu << jnp.uint32(16), jnp.float32) * sc
        bb = lax.bitcast_convert_type(gu & jnp.uint32(0xFFFF0000), jnp.float32) * sc
        m = (jnp.sum(a, axis=1, keepdims=True) + jnp.sum(bb, axis=1, keepdims=True)) / d
        xa = a - m
        xb = bb - m
        var = (
            jnp.sum(xa * xa, axis=1, keepdims=True)
            + jnp.sum(xb * xb, axis=1, keepdims=True)
        ) / d
        inv = lax.rsqrt(var + 1e-5)
        o_ref[:, :half] = x_ref[:, :half] + xa * inv * ga_ref[:, :half] + be_ref[:, :half]
        o_ref[:, half:] = x_ref[:, half:] + xb * inv * ga_ref[:, half:] + be_ref[:, half:]

    in_specs = [
        pl.BlockSpec(memory_space=pltpu.SMEM),
        pl.BlockSpec((_BR, d), lambda i: (block_off + i, 0)),
        pl.BlockSpec((_BR, d // 2), lambda i: (i, 0)),
        pl.BlockSpec((1, d), lambda i: (0, 0)),
        pl.BlockSpec((1, d), lambda i: (0, 0)),
    ]
    args = [scale, xin, gathered, gamma, beta]
    kwargs = {}
    if accum is not None:
        in_specs.append(pl.BlockSpec(memory_space=pl.ANY))
        args.append(accum)
        kwargs["input_output_aliases"] = {5: 0}

    return pl.pallas_call(
        body,
        grid=grid,
        in_specs=in_specs,
        out_specs=pl.BlockSpec((_BR, d), lambda i: (block_off + i, 0)),
        out_shape=jax.ShapeDtypeStruct((n, d), jnp.float32),
        **kwargs,
    )(*args)


def kernel(input_embeddings, position_ids, position_embeddings, pos_scaling, ln_gamma, ln_beta):
    b, s, d = input_embeddings.shape
    n = b * s
    v = position_embeddings.shape[0]
    chunk = n // _K
    pids = jnp.clip(position_ids.astype(jnp.int32), 0, v - 1)
    idx4 = pids.reshape(_K, _NW, chunk // (_NW * _CH), _CH)
    gathered = [_sc_gather_bf16(position_embeddings, idx4[k]) for k in range(_K)]

    xin = input_embeddings.reshape(n, d)
    gamma2 = ln_gamma.reshape(1, d)
    beta2 = ln_beta.reshape(1, d)
    out = None
    blocks_per_chunk = chunk // _BR
    for k in range(_K):
        out = _tc_ln_add_chunk(
            xin, gathered[k], pos_scaling, gamma2, beta2, out, k * blocks_per_chunk
        )
    return out.reshape(b, s, d)


# speedup vs baseline: 1.0080x; 1.0080x over previous
import functools

import jax
import jax.numpy as jnp
from jax import lax
from jax.experimental import pallas as pl
from jax.experimental.pallas import tpu as pltpu
from jax.experimental.pallas import tpu_sc as plsc

_NC = 2
_NS = 16
_NW = _NC * _NS
_CH = 8
_NBUF = 2
_K = 4
_BR = 512


def _sc_gather_bf16(table, idx3):
    nw, nchunks, ch = idx3.shape
    d = table.shape[1]
    n_rows = nw * nchunks * ch
    per_w = nchunks * ch
    n_rounds = nchunks // _NBUF

    @functools.partial(
        pl.kernel,
        mesh=plsc.VectorSubcoreMesh(core_axis_name="c", subcore_axis_name="s"),
        out_type=jax.ShapeDtypeStruct((n_rows, d // 2), jnp.uint32),
        compiler_params=pltpu.CompilerParams(needs_layout_passes=False),
        scratch_types=[
            pltpu.VMEM((nchunks, ch), jnp.int32),
            pltpu.VMEM((ch, d), jnp.float32),
            pltpu.VMEM((ch, d), jnp.float32),
            pltpu.VMEM((ch, d // 2), jnp.uint32),
            pltpu.VMEM((ch, d // 2), jnp.uint32),
            pltpu.SemaphoreType.DMA,
            pltpu.SemaphoreType.DMA,
        ],
    )
    def k(table_hbm, idx_hbm, out_hbm, idx_v, rows0, rows1, o0, o1, sem0, sem1):
        wid = lax.axis_index("s") * _NC + lax.axis_index("c")
        base = wid * per_w
        pltpu.sync_copy(idx_hbm.at[wid], idx_v)
        rows = (rows0, rows1)
        outs = (o0, o1)
        sems = (sem0, sem1)
        half = d // 2

        def convert(src, dst):
            for r in range(ch):

                @plsc.parallel_loop(0, half, step=16, unroll=8)
                def _cv(w):
                    a = plsc.bitcast(src[r, pl.ds(w, 16)], jnp.uint32)
                    b = plsc.bitcast(src[r, pl.ds(half + w, 16)], jnp.uint32)
                    lo = (a + jnp.uint32(0x8000)) >> jnp.uint32(16)
                    hi = (b + jnp.uint32(0x8000)) & jnp.uint32(0xFFFF0000)
                    dst[r, pl.ds(w, 16)] = lo | hi

        for b in range(_NBUF):
            pltpu.async_copy(table_hbm.at[idx_v.at[b]], rows[b], sems[b])

        def round_body(r, carry):
            for b in range(_NBUF):
                c = r * _NBUF + b
                pltpu.make_async_copy(table_hbm.at[idx_v.at[c]], rows[b], sems[b]).wait()
                convert(rows[b], outs[b])
                pltpu.async_copy(table_hbm.at[idx_v.at[c + _NBUF]], rows[b], sems[b])
                pltpu.sync_copy(outs[b], out_hbm.at[pl.ds(base + c * ch, ch)])
            return carry

        lax.fori_loop(0, n_rounds - 1, round_body, 0)
        for b in range(_NBUF):
            c = (n_rounds - 1) * _NBUF + b
            pltpu.make_async_copy(table_hbm.at[idx_v.at[c]], rows[b], sems[b]).wait()
            convert(rows[b], outs[b])
            pltpu.sync_copy(outs[b], out_hbm.at[pl.ds(base + c * ch, ch)])

    return k(table, idx3)


def _tc_ln_add_chunk(xin, gathered, scale, gamma, beta, accum, block_off):
    n, d = xin.shape
    rows = gathered.shape[0]
    grid = (rows // _BR,)

    def body(s_ref, x_ref, g_ref, ga_ref, be_ref, *rest):
        o_ref = rest[-1]
        gu = g_ref[...]
        a = lax.bitcast_convert_type(gu << jnp.uint32(16), jnp.float32)
        bb = lax.bitcast_convert_type(gu & jnp.uint32(0xFFFF0000), jnp.float32)
        x = jnp.concatenate([a, bb], axis=1) * s_ref[0]
        m = jnp.mean(x, axis=1, keepdims=True)
        xc = x - m
        var = jnp.mean(xc * xc, axis=1, keepdims=True)
        inv = lax.rsqrt(var + 1e-5)
        o_ref[...] = x_ref[...] + xc * inv * ga_ref[...] + be_ref[...]

    in_specs = [
        pl.BlockSpec(memory_space=pltpu.SMEM),
        pl.BlockSpec((_BR, d), lambda i: (block_off + i, 0)),
        pl.BlockSpec((_BR, d // 2), lambda i: (i, 0)),
        pl.BlockSpec((1, d), lambda i: (0, 0)),
        pl.BlockSpec((1, d), lambda i: (0, 0)),
    ]
    args = [scale, xin, gathered, gamma, beta]
    kwargs = {}
    if accum is not None:
        in_specs.append(pl.BlockSpec(memory_space=pl.ANY))
        args.append(accum)
        kwargs["input_output_aliases"] = {5: 0}

    return pl.pallas_call(
        body,
        grid=grid,
        in_specs=in_specs,
        out_specs=pl.BlockSpec((_BR, d), lambda i: (block_off + i, 0)),
        out_shape=jax.ShapeDtypeStruct((n, d), jnp.float32),
        **kwargs,
    )(*args)


def kernel(input_embeddings, position_ids, position_embeddings, pos_scaling, ln_gamma, ln_beta):
    b, s, d = input_embeddings.shape
    n = b * s
    v = position_embeddings.shape[0]
    chunk = n // _K
    pids = jnp.clip(position_ids.astype(jnp.int32), 0, v - 1)
    idx4 = pids.reshape(_K, _NW, chunk // (_NW * _CH), _CH)
    gathered = [_sc_gather_bf16(position_embeddings, idx4[k]) for k in range(_K)]

    xin = input_embeddings.reshape(n, d)
    gamma2 = ln_gamma.reshape(1, d)
    beta2 = ln_beta.reshape(1, d)
    out = None
    blocks_per_chunk = chunk // _BR
    for k in range(_K):
        out = _tc_ln_add_chunk(
            xin, gathered[k], pos_scaling, gamma2, beta2, out, k * blocks_per_chunk
        )
    return out.reshape(b, s, d)
